# baseline (speedup 1.0000x reference)
import functools

import jax
import jax.numpy as jnp
from jax import lax
from jax.experimental import pallas as pl
from jax.experimental.pallas import tpu as pltpu
from jax.experimental.pallas import tpu_sc as plsc

_DIM = 300
_DIMP = 384
_SP = 56
_NW = 32
_CH = 128


def _gather(table_hbm, idx_v, buf, sem, j):
    ii = idx_v.at[pl.ds(j * _CH, _CH)]
    pltpu.async_copy(table_hbm.at[ii], buf, sem)


def _wait_gather(table_hbm, idx_v, buf, sem, j):
    ii = idx_v.at[pl.ds(j * _CH, _CH)]
    pltpu.make_async_copy(table_hbm.at[ii], buf, sem).wait()


def _body(idx_hbm, table_hbm, out_hbm, idx_v, buf0, buf1, sem0, sem1):
    per_w = idx_hbm.shape[0] // _NW
    nch = per_w // _CH
    c = lax.axis_index("c")
    s = lax.axis_index("s")
    wid = s * 2 + c
    base = wid * per_w

    pltpu.sync_copy(idx_hbm.at[pl.ds(base, per_w)], idx_v)

    bufs = (buf0, buf1)
    sems = (sem0, sem1)

    _gather(table_hbm, idx_v, buf0, sem0, 0)
    _gather(table_hbm, idx_v, buf1, sem1, 1)

    @pl.loop(0, nch - 2, step=2)
    def _(jj):
        for b in range(2):
            j = jj + b
            _wait_gather(table_hbm, idx_v, bufs[b], sems[b], j)
            pltpu.sync_copy(
                bufs[b], out_hbm.at[pl.ds(base + j * _CH, _CH), :]
            )
            _gather(table_hbm, idx_v, bufs[b], sems[b], j + 2)

    for b in range(2):
        j = nch - 2 + b
        _wait_gather(table_hbm, idx_v, bufs[b], sems[b], j)
        pltpu.sync_copy(bufs[b], out_hbm.at[pl.ds(base + j * _CH, _CH), :])


@jax.jit
def _lookup(idx, table):
    total = idx.shape[0]
    per_w = total // _NW
    mesh = plsc.VectorSubcoreMesh(core_axis_name="c", subcore_axis_name="s")
    f = pl.kernel(
        _body,
        out_type=jax.ShapeDtypeStruct((total, _DIMP), jnp.float32),
        mesh=mesh,
        scratch_types=[
            pltpu.VMEM((per_w,), jnp.int32),
            pltpu.VMEM((_CH, _DIMP), jnp.float32),
            pltpu.VMEM((_CH, _DIMP), jnp.float32),
            pltpu.SemaphoreType.DMA,
            pltpu.SemaphoreType.DMA,
        ],
    )
    return f(idx, table)


def kernel(inp, table):
    b, s = inp.shape
    table_p = jnp.pad(table, ((0, 0), (0, _DIMP - _DIM)))
    npad = _SP - s
    pads = (jnp.arange(b * npad, dtype=jnp.int32) * 521) % table.shape[0]
    idx = jnp.concatenate(
        [inp, pads.reshape(b, npad)], axis=1
    ).reshape(b * _SP)
    y = _lookup(idx, table_p)
    return y.reshape(b, _SP, _DIMP)[:, :s, :_DIM]


# speedup vs baseline: 1.5911x; 1.5911x over previous
import functools

import jax
import jax.numpy as jnp
from jax import lax
from jax.experimental import pallas as pl
from jax.experimental.pallas import tpu as pltpu
from jax.experimental.pallas import tpu_sc as plsc

_DIM = 300
_DIMP = 384
_SP = 56
_NW = 32
_CH = 128


def _gather(table_hbm, tail_hbm, idx_v, buf, sem, j):
    ii = idx_v.at[pl.ds(j * _CH, _CH)]
    pltpu.async_copy(
        table_hbm.at[ii, pl.ds(0, 256)], buf.at[:, pl.ds(0, 256)], sem
    )
    pltpu.async_copy(tail_hbm.at[ii], buf.at[:, pl.ds(256, 128)], sem)


def _wait_gather(table_hbm, tail_hbm, idx_v, buf, sem, j):
    ii = idx_v.at[pl.ds(j * _CH, _CH)]
    pltpu.make_async_copy(
        table_hbm.at[ii, pl.ds(0, 256)], buf.at[:, pl.ds(0, 256)], sem
    ).wait()
    pltpu.make_async_copy(
        tail_hbm.at[ii], buf.at[:, pl.ds(256, 128)], sem
    ).wait()


def _body(idx_hbm, table_hbm, tail_hbm, out_hbm, idx_v, buf0, buf1, sem0, sem1):
    per_w = idx_hbm.shape[0] // _NW
    nch = per_w // _CH
    c = lax.axis_index("c")
    s = lax.axis_index("s")
    wid = s * 2 + c
    base = wid * per_w

    pltpu.sync_copy(idx_hbm.at[pl.ds(base, per_w)], idx_v)

    bufs = (buf0, buf1)
    sems = (sem0, sem1)

    _gather(table_hbm, tail_hbm, idx_v, buf0, sem0, 0)
    _gather(table_hbm, tail_hbm, idx_v, buf1, sem1, 1)

    @pl.loop(0, nch - 2, step=2)
    def _(jj):
        for b in range(2):
            j = jj + b
            _wait_gather(table_hbm, tail_hbm, idx_v, bufs[b], sems[b], j)
            pltpu.sync_copy(
                bufs[b], out_hbm.at[pl.ds(base + j * _CH, _CH), :]
            )
            _gather(table_hbm, tail_hbm, idx_v, bufs[b], sems[b], j + 2)

    for b in range(2):
        j = nch - 2 + b
        _wait_gather(table_hbm, tail_hbm, idx_v, bufs[b], sems[b], j)
        pltpu.sync_copy(bufs[b], out_hbm.at[pl.ds(base + j * _CH, _CH), :])


@jax.jit
def _lookup(idx, table, tail):
    total = idx.shape[0]
    per_w = total // _NW
    mesh = plsc.VectorSubcoreMesh(core_axis_name="c", subcore_axis_name="s")
    f = pl.kernel(
        _body,
        out_type=jax.ShapeDtypeStruct((total, _DIMP), jnp.float32),
        mesh=mesh,
        scratch_types=[
            pltpu.VMEM((per_w,), jnp.int32),
            pltpu.VMEM((_CH, _DIMP), jnp.float32),
            pltpu.VMEM((_CH, _DIMP), jnp.float32),
            pltpu.SemaphoreType.DMA,
            pltpu.SemaphoreType.DMA,
        ],
    )
    return f(idx, table, tail)


def kernel(inp, table):
    b, s = inp.shape
    tail = jnp.pad(
        lax.slice(table, (0, 256), (table.shape[0], _DIM)),
        ((0, 0), (0, 128 - (_DIM - 256))),
    )
    npad = _SP - s
    pads = (jnp.arange(b * npad, dtype=jnp.int32) * 521) % table.shape[0]
    idx = jnp.concatenate(
        [inp, pads.reshape(b, npad)], axis=1
    ).reshape(b * _SP)
    y = _lookup(idx, table, tail)
    return y.reshape(b, _SP, _DIMP)[:, :s, :_DIM]
